# Initial kernel scaffold; baseline (speedup 1.0000x reference)
import jax
import jax.numpy as jnp
from jax.experimental import pallas as pl


def kernel(x, weight, bias, batch):
    raise NotImplementedError("write your pallas kernel here")


# speedup vs baseline: 6.7128x; 6.7128x over previous
import functools

import jax
import jax.numpy as jnp
from jax.experimental import pallas as pl

N = 100000
D = 128
B = 512
EPS = 1e-05
R = 1000


def _pass1_body(x_ref, brow_ref, cnt_ref, s_ref, q_ref):
    i = pl.program_id(0)
    x = x_ref[...]
    rs = jnp.sum(x, axis=1, keepdims=True)
    rq = jnp.sum(x * x, axis=1, keepdims=True)
    b_row = brow_ref[0]
    seg_ids = jax.lax.broadcasted_iota(jnp.int32, (B, R), 0)
    ohT = (seg_ids == b_row).astype(jnp.float32)

    cnt_p = jnp.sum(ohT, axis=1, keepdims=True)
    s_p = jax.lax.dot(ohT, rs, preferred_element_type=jnp.float32)
    q_p = jax.lax.dot(ohT, rq, preferred_element_type=jnp.float32)

    @pl.when(i == 0)
    def _init():
        cnt_ref[...] = jnp.zeros_like(cnt_ref)
        s_ref[...] = jnp.zeros_like(s_ref)
        q_ref[...] = jnp.zeros_like(q_ref)

    cnt_ref[...] += cnt_p
    s_ref[...] += s_p
    q_ref[...] += q_p


def _pass2_body(x_ref, bcol_ref, w_ref, bias_ref, cnt_ref, s_ref, q_ref,
                out_ref):
    x = x_ref[...]
    b_col = bcol_ref[...]
    seg_ids = jax.lax.broadcasted_iota(jnp.int32, (R, B), 1)
    oh = (seg_ids == b_col).astype(jnp.float32)

    g_c = jax.lax.dot(oh, cnt_ref[...], preferred_element_type=jnp.float32)
    g_s = jax.lax.dot(oh, s_ref[...], preferred_element_type=jnp.float32)
    g_q = jax.lax.dot(oh, q_ref[...], preferred_element_type=jnp.float32)

    norm = jnp.maximum(g_c, 1.0) * float(D)
    mean = g_s / norm
    var = jnp.maximum(g_q / norm - mean * mean, 0.0)
    inv = 1.0 / (jnp.sqrt(var) + EPS)
    out_ref[...] = (x - mean) * inv * w_ref[...] + bias_ref[...]


@jax.jit
def kernel(x, weight, bias, batch):
    b32 = batch.astype(jnp.int32)
    brow = b32.reshape(N // R, 1, R)
    bcol = b32.reshape(N, 1)
    w2 = weight.reshape(1, D)
    bias2 = bias.reshape(1, D)
    grid = N // R

    stat_shape = jax.ShapeDtypeStruct((B, 1), jnp.float32)
    cnt, s, q = pl.pallas_call(
        _pass1_body,
        grid=(grid,),
        in_specs=[
            pl.BlockSpec((R, D), lambda i: (i, 0)),
            pl.BlockSpec((1, 1, R), lambda i: (i, 0, 0)),
        ],
        out_specs=[
            pl.BlockSpec((B, 1), lambda i: (0, 0)),
            pl.BlockSpec((B, 1), lambda i: (0, 0)),
            pl.BlockSpec((B, 1), lambda i: (0, 0)),
        ],
        out_shape=[stat_shape, stat_shape, stat_shape],
    )(x, brow)

    out = pl.pallas_call(
        _pass2_body,
        grid=(grid,),
        in_specs=[
            pl.BlockSpec((R, D), lambda i: (i, 0)),
            pl.BlockSpec((R, 1), lambda i: (i, 0)),
            pl.BlockSpec((1, D), lambda i: (0, 0)),
            pl.BlockSpec((1, D), lambda i: (0, 0)),
            pl.BlockSpec((B, 1), lambda i: (0, 0)),
            pl.BlockSpec((B, 1), lambda i: (0, 0)),
            pl.BlockSpec((B, 1), lambda i: (0, 0)),
        ],
        out_specs=pl.BlockSpec((R, D), lambda i: (i, 0)),
        out_shape=jax.ShapeDtypeStruct((N, D), jnp.float32),
    )(x, bcol, w2, bias2, cnt, s, q)
    return out


# speedup vs baseline: 9.8154x; 1.4622x over previous
import jax
import jax.numpy as jnp
from jax.experimental import pallas as pl

N = 100000
D = 128
B = 512
EPS = 1e-05
R = 2000


def _pass1_body(x_ref, brow_ref, acc_ref, mi_ref):
    i = pl.program_id(0)
    x = x_ref[...]
    rs = jnp.sum(x, axis=1, keepdims=True)
    rq = jnp.sum(x * x, axis=1, keepdims=True)
    ones = jnp.ones((R, 1), jnp.float32)
    vals = jnp.concatenate([ones, rs, rq, jnp.zeros((R, 1), jnp.float32)],
                           axis=1)
    b_row = brow_ref[0]
    seg_ids = jax.lax.broadcasted_iota(jnp.int32, (B, R), 0)
    ohT = (seg_ids == b_row).astype(jnp.float32)

    @pl.when(i == 0)
    def _init():
        acc_ref[...] = jnp.zeros_like(acc_ref)

    acc_ref[...] += jax.lax.dot(ohT, vals, preferred_element_type=jnp.float32)

    @pl.when(i == pl.num_programs(0) - 1)
    def _finalize():
        acc = acc_ref[...]
        cnt = acc[:, 0:1]
        s = acc[:, 1:2]
        q = acc[:, 2:3]
        norm = jnp.maximum(cnt, 1.0) * float(D)
        mean = s / norm
        var = jnp.maximum(q / norm - mean * mean, 0.0)
        inv = 1.0 / (jnp.sqrt(var) + EPS)
        mi_ref[...] = jnp.concatenate([mean, inv], axis=1)


def _pass2_body(x_ref, bcol_ref, w_ref, bias_ref, mi_ref, out_ref):
    x = x_ref[...]
    b_col = bcol_ref[...]
    seg_ids = jax.lax.broadcasted_iota(jnp.int32, (R, B), 1)
    oh = (seg_ids == b_col).astype(jnp.float32)
    g = jax.lax.dot(oh, mi_ref[...], preferred_element_type=jnp.float32)
    mean = g[:, 0:1]
    inv = g[:, 1:2]
    out_ref[...] = (x - mean) * inv * w_ref[...] + bias_ref[...]


@jax.jit
def kernel(x, weight, bias, batch):
    b32 = batch.astype(jnp.int32)
    brow = b32.reshape(N // R, 1, R)
    bcol = b32.reshape(N, 1)
    w2 = weight.reshape(1, D)
    bias2 = bias.reshape(1, D)
    grid = N // R

    acc, mi = pl.pallas_call(
        _pass1_body,
        grid=(grid,),
        in_specs=[
            pl.BlockSpec((R, D), lambda i: (i, 0)),
            pl.BlockSpec((1, 1, R), lambda i: (i, 0, 0)),
        ],
        out_specs=[
            pl.BlockSpec((B, 4), lambda i: (0, 0)),
            pl.BlockSpec((B, 2), lambda i: (0, 0)),
        ],
        out_shape=[
            jax.ShapeDtypeStruct((B, 4), jnp.float32),
            jax.ShapeDtypeStruct((B, 2), jnp.float32),
        ],
    )(x, brow)

    out = pl.pallas_call(
        _pass2_body,
        grid=(grid,),
        in_specs=[
            pl.BlockSpec((R, D), lambda i: (i, 0)),
            pl.BlockSpec((R, 1), lambda i: (i, 0)),
            pl.BlockSpec((1, D), lambda i: (0, 0)),
            pl.BlockSpec((1, D), lambda i: (0, 0)),
            pl.BlockSpec((B, 2), lambda i: (0, 0)),
        ],
        out_specs=pl.BlockSpec((R, D), lambda i: (i, 0)),
        out_shape=jax.ShapeDtypeStruct((N, D), jnp.float32),
    )(x, bcol, w2, bias2, mi)
    return out
